# baseline (speedup 1.0000x reference)
import functools

import jax
import jax.numpy as jnp
import numpy as np
from jax import lax
from jax.experimental import pallas as pl
from jax.experimental.pallas import tpu as pltpu
from jax.experimental.pallas import tpu_sc as plsc

SEQ_LEN = 1024
N_BINS = 64
NTILES = 32
ROWS_PER_TILE = SEQ_LEN // NTILES
WIN = SEQ_LEN + ROWS_PER_TILE
K_CUT = 48
BIG = np.float32(1e30)
DMA_LAG = 4


def _edge_arrays():
    e = np.linspace(2.0, 22.0, N_BINS).astype(np.float32)[1:]
    elo = np.empty((N_BINS,), dtype=np.float32)
    elo[0] = -BIG
    elo[1:] = e
    ehi = np.empty((N_BINS,), dtype=np.float32)
    ehi[:63] = e
    ehi[63] = BIG
    return elo, ehi


def _sc_body(d_hbm, elo_hbm, ehi_hbm, out_hbm, d_v, elo_v, ehi_v, tab, sem):
    cid = lax.axis_index("c")
    sid = lax.axis_index("s")
    wid = sid * 2 + cid
    base = wid * ROWS_PER_TILE
    t0 = (SEQ_LEN - ROWS_PER_TILE) - base

    pltpu.sync_copy(d_hbm, d_v.at[pl.ds(8, SEQ_LEN)])
    pltpu.sync_copy(elo_hbm, elo_v)
    pltpu.sync_copy(ehi_hbm, ehi_v)

    elo_c = [elo_v[pl.ds(c * 16, 16)] for c in range(4)]
    ehi_c = [ehi_v[pl.ds(c * 16, 16)] for c in range(4)]

    lo_acc = [jnp.zeros((16,), jnp.float32) for _ in range(4)]
    hi_acc = [jnp.zeros((16,), jnp.float32) for _ in range(4)]
    for kk in range(K_CUT):
        dkv = plsc.load_gather(d_v, [jnp.full((16,), kk + 8, jnp.int32)])
        for c in range(4):
            lo_acc[c] = lo_acc[c] + (dkv < elo_c[c]).astype(jnp.float32)
            hi_acc[c] = hi_acc[c] + (dkv < ehi_c[c]).astype(jnp.float32)
    rest = jnp.float32(SEQ_LEN - K_CUT)
    for c in range(4):
        hi_acc[c] = hi_acc[c] + jnp.where(
            ehi_c[c] == BIG, rest, jnp.float32(0.0)
        )

    def build_row(t, carry):
        lo0, lo1, lo2, lo3, hi0, hi1, hi2, hi3 = carry
        sep = jnp.abs(t0 + t - (SEQ_LEN - 1)).astype(jnp.float32)
        sepv = jnp.broadcast_to(sep, (16,))
        for c, (lo, hi) in enumerate(
            ((lo0, hi0), (lo1, hi1), (lo2, hi2), (lo3, hi3))
        ):
            cond = (sepv >= lo) & (sepv < hi)
            tab[t, pl.ds(c * 16, 16)] = jnp.where(
                cond, jnp.float32(10.0), jnp.float32(-10.0)
            )
        return carry
    lax.fori_loop(0, WIN, build_row, tuple(lo_acc) + tuple(hi_acc))

    def _copy(j):
        return pltpu.make_async_copy(
            tab.at[pl.ds((ROWS_PER_TILE - 1) - j, SEQ_LEN), :],
            out_hbm.at[0, base + j],
            sem,
        )

    for j in range(ROWS_PER_TILE):
        _copy(j).start()
        if j >= DMA_LAG:
            _copy(j - DMA_LAG).wait()
    for j in range(ROWS_PER_TILE - DMA_LAG, ROWS_PER_TILE):
        _copy(j).wait()


@jax.jit
def _logits(d, elo, ehi):
    mesh = plsc.VectorSubcoreMesh(core_axis_name="c", subcore_axis_name="s")
    run = functools.partial(
        pl.kernel,
        mesh=mesh,
        out_type=jax.ShapeDtypeStruct(
            (1, SEQ_LEN, SEQ_LEN, N_BINS), jnp.float32
        ),
        scratch_types=[
            pltpu.VMEM((SEQ_LEN + 8,), jnp.float32),
            pltpu.VMEM((N_BINS,), jnp.float32),
            pltpu.VMEM((N_BINS,), jnp.float32),
            pltpu.VMEM((WIN, N_BINS), jnp.float32),
            pltpu.SemaphoreType.DMA,
        ],
        compiler_params=pltpu.CompilerParams(
            needs_layout_passes=False, use_tc_tiling_on_sc=False
        ),
    )(_sc_body)
    return run(d, elo, ehi)


def kernel(x):
    B, L, _ = x.shape
    k = jnp.arange(SEQ_LEN + 2, dtype=jnp.float32)
    sep_to_dist = jnp.clip(2.0 + 2.5 * jnp.power(k, 0.55), 2.0, 22.0)
    elo, ehi = _edge_arrays()
    return _logits(sep_to_dist[:SEQ_LEN], jnp.asarray(elo), jnp.asarray(ehi))


# speedup vs baseline: 1.7514x; 1.7514x over previous
import functools

import jax
import jax.numpy as jnp
import numpy as np
from jax import lax
from jax.experimental import pallas as pl
from jax.experimental.pallas import tpu as pltpu
from jax.experimental.pallas import tpu_sc as plsc

SEQ_LEN = 1024
N_BINS = 64
NTILES = 32
ROWS_PER_TILE = SEQ_LEN // NTILES
BAND = 256
MAX_OFF = 8 * (ROWS_PER_TILE - 1)
WIN = SEQ_LEN + MAX_OFF + 8
K_CUT = 48
BIG = np.float32(1e30)
DMA_LAG = 8
OFF = 8


def _edge_arrays():
    e = np.linspace(2.0, 22.0, N_BINS).astype(np.float32)[1:]
    elo = np.empty((N_BINS,), dtype=np.float32)
    elo[0] = -BIG
    elo[1:] = e
    ehi = np.empty((N_BINS,), dtype=np.float32)
    ehi[:63] = e
    ehi[63] = BIG
    return elo, ehi


def _sc_body(d_hbm, elo_hbm, ehi_hbm, out_hbm, d_v, elo_v, ehi_v, lohi_v,
             tab, sem):
    cid = lax.axis_index("c")
    sid = lax.axis_index("s")
    wid = sid * 2 + cid
    band = wid // 8
    r = wid % 8
    row0 = BAND * band + r
    t0 = (SEQ_LEN - 1) - row0 - MAX_OFF

    pltpu.sync_copy(d_hbm, d_v.at[pl.ds(OFF, SEQ_LEN)])
    pltpu.sync_copy(elo_hbm, elo_v)
    pltpu.sync_copy(ehi_hbm, ehi_v)

    elo_c = [elo_v[pl.ds(c * 16, 16)] for c in range(4)]
    ehi_c = [ehi_v[pl.ds(c * 16, 16)] for c in range(4)]

    lo_acc = [jnp.zeros((16,), jnp.float32) for _ in range(4)]
    hi_acc = [jnp.zeros((16,), jnp.float32) for _ in range(4)]
    for kk in range(K_CUT):
        dkv = plsc.load_gather(d_v, [jnp.full((16,), kk + OFF, jnp.int32)])
        for c in range(4):
            lo_acc[c] = lo_acc[c] + (dkv < elo_c[c]).astype(jnp.float32)
            hi_acc[c] = hi_acc[c] + (dkv < ehi_c[c]).astype(jnp.float32)
    rest = jnp.float32(SEQ_LEN - K_CUT)
    for c in range(4):
        hi_acc[c] = hi_acc[c] + jnp.where(
            ehi_c[c] == BIG, rest, jnp.float32(0.0)
        )
        lohi_v[pl.ds(OFF + c * 16, 16)] = lo_acc[c]
        lohi_v[pl.ds(OFF + 64 + c * 16, 16)] = hi_acc[c]

    tchunks = WIN // 16
    lane = lax.broadcasted_iota(jnp.int32, (16,), 0)
    for b in range(N_BINS):
        lo_b = plsc.load_gather(lohi_v, [jnp.full((16,), OFF + b, jnp.int32)])
        hi_b = plsc.load_gather(
            lohi_v, [jnp.full((16,), OFF + 64 + b, jnp.int32)]
        )

        def build_chunk(tc, carry, lo_b=lo_b, hi_b=hi_b, b=b):
            sep = jnp.abs(
                lane + (t0 + tc * 16 - (SEQ_LEN - 1))
            ).astype(jnp.float32)
            cond = (sep >= lo_b) & (sep < hi_b)
            tab[b, pl.ds(tc * 16, 16)] = jnp.where(
                cond, jnp.float32(10.0), jnp.float32(-10.0)
            )
            return carry

        lax.fori_loop(0, tchunks, build_chunk, 0)

    def _copy(m):
        return pltpu.make_async_copy(
            tab.at[:, pl.ds(MAX_OFF - 8 * m, SEQ_LEN)],
            out_hbm.at[0, row0 + 8 * m],
            sem,
        )

    for j in range(ROWS_PER_TILE):
        _copy(j).start()
        if j >= DMA_LAG:
            _copy(j - DMA_LAG).wait()
    for j in range(ROWS_PER_TILE - DMA_LAG, ROWS_PER_TILE):
        _copy(j).wait()


@jax.jit
def _logits(d, elo, ehi):
    mesh = plsc.VectorSubcoreMesh(core_axis_name="c", subcore_axis_name="s")
    run = functools.partial(
        pl.kernel,
        mesh=mesh,
        out_type=jax.ShapeDtypeStruct(
            (1, SEQ_LEN, N_BINS, SEQ_LEN), jnp.float32
        ),
        scratch_types=[
            pltpu.VMEM((SEQ_LEN + OFF,), jnp.float32),
            pltpu.VMEM((N_BINS,), jnp.float32),
            pltpu.VMEM((N_BINS,), jnp.float32),
            pltpu.VMEM((OFF + 2 * N_BINS,), jnp.float32),
            pltpu.VMEM((N_BINS, WIN), jnp.float32),
            pltpu.SemaphoreType.DMA,
        ],
        compiler_params=pltpu.CompilerParams(
            needs_layout_passes=False, use_tc_tiling_on_sc=False
        ),
    )(_sc_body)
    return jnp.swapaxes(run(d, elo, ehi), 2, 3)


def kernel(x):
    B, L, _ = x.shape
    k = jnp.arange(SEQ_LEN + 2, dtype=jnp.float32)
    sep_to_dist = jnp.clip(2.0 + 2.5 * jnp.power(k, 0.55), 2.0, 22.0)
    elo, ehi = _edge_arrays()
    return _logits(sep_to_dist[:SEQ_LEN], jnp.asarray(elo), jnp.asarray(ehi))


# speedup vs baseline: 1.8273x; 1.0433x over previous
import functools

import jax
import jax.numpy as jnp
import numpy as np
from jax import lax
from jax.experimental import pallas as pl
from jax.experimental.pallas import tpu as pltpu
from jax.experimental.pallas import tpu_sc as plsc

SEQ_LEN = 1024
N_BINS = 64
NTILES = 32
ROWS_PER_TILE = SEQ_LEN // NTILES
BAND = 256
MAX_OFF = 8 * (ROWS_PER_TILE - 1)
WIN = SEQ_LEN + MAX_OFF + 8
K_CUT = 48
BIG = np.float32(1e30)
DMA_LAG = 16
OFF = 8


def _edge_arrays():
    e = np.linspace(2.0, 22.0, N_BINS).astype(np.float32)[1:]
    elo = np.empty((N_BINS,), dtype=np.float32)
    elo[0] = -BIG
    elo[1:] = e
    ehi = np.empty((N_BINS,), dtype=np.float32)
    ehi[:63] = e
    ehi[63] = BIG
    return elo, ehi


def _sc_body(d_hbm, elo_hbm, ehi_hbm, out_hbm, d_v, elo_v, ehi_v, lohi_v,
             tab, sem):
    cid = lax.axis_index("c")
    sid = lax.axis_index("s")
    wid = sid * 2 + cid
    band = wid // 8
    r = wid % 8
    row0 = BAND * band + r
    t0 = (SEQ_LEN - 1) - row0 - MAX_OFF

    pltpu.sync_copy(d_hbm, d_v.at[pl.ds(OFF, SEQ_LEN)])
    pltpu.sync_copy(elo_hbm, elo_v)
    pltpu.sync_copy(ehi_hbm, ehi_v)

    elo_c = [elo_v[pl.ds(c * 16, 16)] for c in range(4)]
    ehi_c = [ehi_v[pl.ds(c * 16, 16)] for c in range(4)]

    lo_acc = [jnp.zeros((16,), jnp.float32) for _ in range(4)]
    hi_acc = [jnp.zeros((16,), jnp.float32) for _ in range(4)]
    for kk in range(K_CUT):
        dkv = plsc.load_gather(d_v, [jnp.full((16,), kk + OFF, jnp.int32)])
        for c in range(4):
            lo_acc[c] = lo_acc[c] + (dkv < elo_c[c]).astype(jnp.float32)
            hi_acc[c] = hi_acc[c] + (dkv < ehi_c[c]).astype(jnp.float32)
    rest = jnp.float32(SEQ_LEN - K_CUT)
    for c in range(4):
        hi_acc[c] = hi_acc[c] + jnp.where(
            ehi_c[c] == BIG, rest, jnp.float32(0.0)
        )
        lohi_v[pl.ds(OFF + c * 16, 16)] = lo_acc[c]
        lohi_v[pl.ds(OFF + 64 + c * 16, 16)] = hi_acc[c]

    tchunks = WIN // 16
    lane = lax.broadcasted_iota(jnp.int32, (16,), 0)
    t_star = (SEQ_LEN - 1) - t0
    neg = jnp.full((16,), jnp.float32(-10.0))
    lo_last = plsc.load_gather(
        lohi_v, [jnp.full((16,), OFF + (N_BINS - 1), jnp.int32)]
    )
    hi_last = plsc.load_gather(
        lohi_v, [jnp.full((16,), OFF + 64 + (N_BINS - 1), jnp.int32)]
    )

    def init_chunk(tc, carry):
        sep = jnp.abs(lane + (tc * 16 - t_star)).astype(jnp.float32)
        cond = (sep >= lo_last) & (sep < hi_last)
        tab[N_BINS - 1, pl.ds(tc * 16, 16)] = jnp.where(
            cond, jnp.float32(10.0), jnp.float32(-10.0)
        )
        for b in range(N_BINS - 1):
            tab[b, pl.ds(tc * 16, 16)] = neg
        return carry

    lax.fori_loop(0, tchunks, init_chunk, 0)

    c_lo = (t_star - 43) // 16
    c_hi = (t_star + 43) // 16 + 1

    def band_chunk(tc, carry):
        sep = jnp.abs(lane + (tc * 16 - t_star)).astype(jnp.float32)
        for b in range(N_BINS - 1):
            lo_b = plsc.load_gather(
                lohi_v, [jnp.full((16,), OFF + b, jnp.int32)]
            )
            hi_b = plsc.load_gather(
                lohi_v, [jnp.full((16,), OFF + 64 + b, jnp.int32)]
            )
            cond = (sep >= lo_b) & (sep < hi_b)
            tab[b, pl.ds(tc * 16, 16)] = jnp.where(
                cond, jnp.float32(10.0), jnp.float32(-10.0)
            )
        return carry

    lax.fori_loop(c_lo, c_hi, band_chunk, 0)

    def _copy(m):
        return pltpu.make_async_copy(
            tab.at[:, pl.ds(MAX_OFF - 8 * m, SEQ_LEN)],
            out_hbm.at[0, row0 + 8 * m],
            sem,
        )

    for j in range(ROWS_PER_TILE):
        _copy(j).start()
        if j >= DMA_LAG:
            _copy(j - DMA_LAG).wait()
    for j in range(ROWS_PER_TILE - DMA_LAG, ROWS_PER_TILE):
        _copy(j).wait()


@jax.jit
def _logits(d, elo, ehi):
    mesh = plsc.VectorSubcoreMesh(core_axis_name="c", subcore_axis_name="s")
    run = functools.partial(
        pl.kernel,
        mesh=mesh,
        out_type=jax.ShapeDtypeStruct(
            (1, SEQ_LEN, N_BINS, SEQ_LEN), jnp.float32
        ),
        scratch_types=[
            pltpu.VMEM((SEQ_LEN + OFF,), jnp.float32),
            pltpu.VMEM((N_BINS,), jnp.float32),
            pltpu.VMEM((N_BINS,), jnp.float32),
            pltpu.VMEM((OFF + 2 * N_BINS,), jnp.float32),
            pltpu.VMEM((N_BINS, WIN), jnp.float32),
            pltpu.SemaphoreType.DMA,
        ],
        compiler_params=pltpu.CompilerParams(
            needs_layout_passes=False, use_tc_tiling_on_sc=False
        ),
    )(_sc_body)
    return jnp.swapaxes(run(d, elo, ehi), 2, 3)


def kernel(x):
    B, L, _ = x.shape
    k = jnp.arange(SEQ_LEN + 2, dtype=jnp.float32)
    sep_to_dist = jnp.clip(2.0 + 2.5 * jnp.power(k, 0.55), 2.0, 22.0)
    elo, ehi = _edge_arrays()
    return _logits(sep_to_dist[:SEQ_LEN], jnp.asarray(elo), jnp.asarray(ehi))


# speedup vs baseline: 8.1022x; 4.4341x over previous
import functools

import jax
import jax.numpy as jnp
import numpy as np
from jax.experimental import pallas as pl

SEQ_LEN = 1024
N_BINS = 64
RB = 32
BIG = np.float32(1e30)


def _edges_cols():
    e = np.linspace(2.0, 22.0, N_BINS).astype(np.float32)[1:]
    elo = np.empty((N_BINS,), dtype=np.float32)
    elo[0] = -BIG
    elo[1:] = e
    ehi = np.empty((N_BINS,), dtype=np.float32)
    ehi[:63] = e
    ehi[63] = BIG
    elo_p = np.repeat(elo[:, None], 128, axis=1)
    ehi_p = np.repeat(ehi[:, None], 128, axis=1)
    return elo_p, ehi_p


def _body(lut_ref, elo_ref, ehi_ref, out_ref):
    d_row = lut_ref[0:1, :]
    elo_col = elo_ref[:, 0:1]
    ehi_col = ehi_ref[:, 0:1]
    lo = jnp.sum((d_row < elo_col).astype(jnp.float32), axis=1, keepdims=True)
    hi = jnp.sum((d_row < ehi_col).astype(jnp.float32), axis=1, keepdims=True)
    lo4 = lo.reshape(1, 1, N_BINS, 1)
    hi4 = hi.reshape(1, 1, N_BINS, 1)

    r = pl.program_id(0)
    row = jax.lax.broadcasted_iota(jnp.int32, (1, RB, 1, SEQ_LEN), 1)
    col = jax.lax.broadcasted_iota(jnp.int32, (1, RB, 1, SEQ_LEN), 3)
    sep = jnp.abs(row + (r * RB) - col).astype(jnp.float32)
    cond = (sep >= lo4) & (sep < hi4)
    out_ref[...] = jnp.where(cond, jnp.float32(10.0), jnp.float32(-10.0))


@jax.jit
def _logits(lut, elo_p, ehi_p):
    out = pl.pallas_call(
        _body,
        grid=(SEQ_LEN // RB,),
        in_specs=[
            pl.BlockSpec((8, SEQ_LEN), lambda i: (0, 0)),
            pl.BlockSpec((N_BINS, 128), lambda i: (0, 0)),
            pl.BlockSpec((N_BINS, 128), lambda i: (0, 0)),
        ],
        out_specs=pl.BlockSpec((1, RB, N_BINS, SEQ_LEN), lambda i: (0, i, 0, 0)),
        out_shape=jax.ShapeDtypeStruct(
            (1, SEQ_LEN, N_BINS, SEQ_LEN), jnp.float32
        ),
    )(lut, elo_p, ehi_p)
    return jnp.swapaxes(out, 2, 3)


def kernel(x):
    B, L, _ = x.shape
    k = jnp.arange(SEQ_LEN + 2, dtype=jnp.float32)
    sep_to_dist = jnp.clip(2.0 + 2.5 * jnp.power(k, 0.55), 2.0, 22.0)
    lut8 = jnp.broadcast_to(sep_to_dist[None, :SEQ_LEN], (8, SEQ_LEN))
    elo_p, ehi_p = _edges_cols()
    return _logits(lut8, jnp.asarray(elo_p), jnp.asarray(ehi_p))
